# Initial kernel scaffold; baseline (speedup 1.0000x reference)
import jax
import jax.numpy as jnp
from jax.experimental import pallas as pl


def kernel(f1, f2, pre1, pre2, fea1, fea2):
    raise NotImplementedError("write your pallas kernel here")


# speedup vs baseline: 2.0618x; 2.0618x over previous
import jax
import jax.numpy as jnp
from jax.experimental import pallas as pl
from jax.experimental.pallas import tpu as pltpu

_HALF = 6.0
_MAXC = 63.0
_M_EMA = 0.999
_B, _C, _H, _W, _L = 64, 256, 64, 64, 21
_CORES = 2
_BPC = _B // _CORES


def _masks_and_inv_s(pre_blk):
    x = pre_blk[0, :, 0:1]
    y = pre_blk[0, :, 1:2]
    down = jnp.maximum(y - _HALF, 0.0).astype(jnp.int32)
    left = jnp.maximum(x - _HALF, 0.0).astype(jnp.int32)
    upper = jnp.minimum(y + _HALF, _MAXC).astype(jnp.int32)
    right = jnp.minimum(x + _HALF, _MAXC).astype(jnp.int32)
    jw = jax.lax.broadcasted_iota(jnp.int32, (_L, _W), 1)
    colmask = ((jw >= down) & (jw < upper)).astype(jnp.float32)
    jh = jax.lax.broadcasted_iota(jnp.int32, (_H, _L), 0)
    rowmask_t = ((jh >= left.T) & (jh < right.T)).astype(jnp.float32)
    s = ((upper - down + 1) * (right - left + 1)).astype(jnp.float32)
    return colmask, rowmask_t, 1.0 / s


def _pool_one(f_blk, pre_blk):
    colmask, rowmask_t, inv_s = _masks_and_inv_s(pre_blk)
    feat = f_blk[0].reshape(_C * _H, _W)
    t = jax.lax.dot_general(
        feat, colmask, (((1,), (1,)), ((), ())),
        preferred_element_type=jnp.float32)
    t3 = t.reshape(_C, _H, _L)
    pooled = jnp.sum(t3 * rowmask_t[None], axis=1)
    return pooled * inv_s.T


def _pool_kernel(f1_ref, f2_ref, pre1_ref, pre2_ref, o1_ref, o2_ref):
    i = pl.program_id(1)

    @pl.when(i == 0)
    def _():
        o1_ref[...] = jnp.zeros_like(o1_ref)
        o2_ref[...] = jnp.zeros_like(o2_ref)

    o1_ref[...] += _pool_one(f1_ref, pre1_ref)[None]
    o2_ref[...] += _pool_one(f2_ref, pre2_ref)[None]


def _finalize_kernel(p1_ref, p2_ref, fea1_ref, fea2_ref, o_ref):
    m1 = (p1_ref[0] + p1_ref[1]) * (1.0 / _B)
    m2 = (p2_ref[0] + p2_ref[1]) * (1.0 / _B)
    fea_c1 = _M_EMA * m1 + (1.0 - _M_EMA) * fea1_ref[...]
    fea_c2 = _M_EMA * m2 + (1.0 - _M_EMA) * fea2_ref[...]
    z = fea_c1 - jnp.max(fea_c1, axis=0, keepdims=True)
    log_p = z - jnp.log(jnp.sum(jnp.exp(z), axis=0, keepdims=True))
    q = fea_c2 / jnp.sum(fea_c2, axis=0, keepdims=True)
    kl = jnp.where(q > 0, q * (jnp.log(jnp.where(q > 0, q, 1.0)) - log_p), 0.0)
    o_ref[0, 0] = jnp.sum(kl) * (1.0 / _L)


def kernel(f1, f2, pre1, pre2, fea1, fea2):
    f_spec = pl.BlockSpec((1, _C, _H, _W), lambda k, i: (k * _BPC + i, 0, 0, 0))
    pre_spec = pl.BlockSpec((1, _L, 2), lambda k, i: (k * _BPC + i, 0, 0))
    out_spec = pl.BlockSpec((1, _C, _L), lambda k, i: (k, 0, 0))

    p1, p2 = pl.pallas_call(
        _pool_kernel,
        grid=(_CORES, _BPC),
        in_specs=[f_spec, f_spec, pre_spec, pre_spec],
        out_specs=[out_spec, out_spec],
        out_shape=[jax.ShapeDtypeStruct((_CORES, _C, _L), jnp.float32)] * 2,
        compiler_params=pltpu.CompilerParams(
            dimension_semantics=("parallel", "arbitrary")),
        name="window_pool",
    )(f1, f2, pre1, pre2)

    out = pl.pallas_call(
        _finalize_kernel,
        in_specs=[pl.BlockSpec(memory_space=pltpu.VMEM)] * 4,
        out_specs=pl.BlockSpec(memory_space=pltpu.SMEM),
        out_shape=jax.ShapeDtypeStruct((1, 1), jnp.float32),
        name="ema_kl_finalize",
    )(p1, p2, fea1.T, fea2.T)
    return out[0, 0]


# speedup vs baseline: 3.9650x; 1.9231x over previous
import jax
import jax.numpy as jnp
from jax.experimental import pallas as pl
from jax.experimental.pallas import tpu as pltpu

_HALF = 6.0
_MAXC = 63.0
_M_EMA = 0.999
_B, _C, _H, _W, _L = 64, 256, 64, 64, 21
_CORES = 2
_BPC = _B // _CORES


def _pool_one(f_blk, pre_blk):
    x = pre_blk[0, :, 0:1]
    y = pre_blk[0, :, 1:2]
    down = jnp.maximum(y - _HALF, 0.0).astype(jnp.int32)
    left = jnp.maximum(x - _HALF, 0.0).astype(jnp.int32)
    upper = jnp.minimum(y + _HALF, _MAXC).astype(jnp.int32)
    right = jnp.minimum(x + _HALF, _MAXC).astype(jnp.int32)
    j = jax.lax.broadcasted_iota(jnp.int32, (_L, _H * _W), 1)
    h = j >> 6
    w = j & (_W - 1)
    mask = ((h >= left) & (h < right) & (w >= down) & (w < upper)
            ).astype(jnp.float32)
    s = ((upper - down + 1) * (right - left + 1)).astype(jnp.float32)
    pooled = jax.lax.dot_general(
        f_blk[0], mask, (((1,), (1,)), ((), ())),
        preferred_element_type=jnp.float32)
    return pooled * (1.0 / s).T


def _pool_kernel(f1_ref, f2_ref, pre1_ref, pre2_ref, o1_ref, o2_ref):
    i = pl.program_id(1)

    @pl.when(i == 0)
    def _():
        o1_ref[...] = jnp.zeros_like(o1_ref)
        o2_ref[...] = jnp.zeros_like(o2_ref)

    o1_ref[...] += _pool_one(f1_ref, pre1_ref)[None]
    o2_ref[...] += _pool_one(f2_ref, pre2_ref)[None]


def _finalize_kernel(p1_ref, p2_ref, fea1_ref, fea2_ref, o_ref):
    m1 = (p1_ref[0] + p1_ref[1]) * (1.0 / _B)
    m2 = (p2_ref[0] + p2_ref[1]) * (1.0 / _B)
    fea_c1 = _M_EMA * m1 + (1.0 - _M_EMA) * fea1_ref[...]
    fea_c2 = _M_EMA * m2 + (1.0 - _M_EMA) * fea2_ref[...]
    z = fea_c1 - jnp.max(fea_c1, axis=0, keepdims=True)
    log_p = z - jnp.log(jnp.sum(jnp.exp(z), axis=0, keepdims=True))
    q = fea_c2 / jnp.sum(fea_c2, axis=0, keepdims=True)
    kl = jnp.where(q > 0, q * (jnp.log(jnp.where(q > 0, q, 1.0)) - log_p), 0.0)
    o_ref[0, 0] = jnp.sum(kl) * (1.0 / _L)


def kernel(f1, f2, pre1, pre2, fea1, fea2):
    f_spec = pl.BlockSpec((1, _C, _H * _W), lambda k, i: (k * _BPC + i, 0, 0))
    pre_spec = pl.BlockSpec((1, _L, 2), lambda k, i: (k * _BPC + i, 0, 0))
    out_spec = pl.BlockSpec((1, _C, _L), lambda k, i: (k, 0, 0))

    p1, p2 = pl.pallas_call(
        _pool_kernel,
        grid=(_CORES, _BPC),
        in_specs=[f_spec, f_spec, pre_spec, pre_spec],
        out_specs=[out_spec, out_spec],
        out_shape=[jax.ShapeDtypeStruct((_CORES, _C, _L), jnp.float32)] * 2,
        compiler_params=pltpu.CompilerParams(
            dimension_semantics=("parallel", "arbitrary")),
        name="window_pool",
    )(f1.reshape(_B, _C, _H * _W), f2.reshape(_B, _C, _H * _W), pre1, pre2)

    out = pl.pallas_call(
        _finalize_kernel,
        in_specs=[pl.BlockSpec(memory_space=pltpu.VMEM)] * 4,
        out_specs=pl.BlockSpec(memory_space=pltpu.SMEM),
        out_shape=jax.ShapeDtypeStruct((1, 1), jnp.float32),
        name="ema_kl_finalize",
    )(p1, p2, fea1.T, fea2.T)
    return out[0, 0]


# speedup vs baseline: 15.6821x; 3.9551x over previous
import jax
import jax.numpy as jnp
from jax.experimental import pallas as pl
from jax.experimental.pallas import tpu as pltpu

_HALF = 6.0
_MAXC = 63.0
_M_EMA = 0.999
_B, _C, _H, _W, _L = 64, 256, 64, 64, 21
_CORES = 2
_BPC = _B // _CORES


def _pool_one(f_blk, pre_blk):
    x = pre_blk[0, :, 0:1]
    y = pre_blk[0, :, 1:2]
    down = jnp.maximum(y - _HALF, 0.0).astype(jnp.int32)
    left = jnp.maximum(x - _HALF, 0.0).astype(jnp.int32)
    upper = jnp.minimum(y + _HALF, _MAXC).astype(jnp.int32)
    right = jnp.minimum(x + _HALF, _MAXC).astype(jnp.int32)
    j = jax.lax.broadcasted_iota(jnp.int32, (_L, _H * _W), 1)
    h = j >> 6
    w = j & (_W - 1)
    mask = ((h >= left) & (h < right) & (w >= down) & (w < upper)
            ).astype(jnp.float32)
    s = ((upper - down + 1) * (right - left + 1)).astype(jnp.float32)
    pooled = jnp.dot(mask, f_blk[0], preferred_element_type=jnp.float32)
    return pooled * (1.0 / s)


def _pool_kernel(f1_ref, f2_ref, pre1_ref, pre2_ref, o1_ref, o2_ref):
    i = pl.program_id(1)

    @pl.when(i == 0)
    def _():
        o1_ref[...] = jnp.zeros_like(o1_ref)
        o2_ref[...] = jnp.zeros_like(o2_ref)

    o1_ref[...] += _pool_one(f1_ref, pre1_ref)[None]
    o2_ref[...] += _pool_one(f2_ref, pre2_ref)[None]


def _finalize_kernel(p1_ref, p2_ref, fea1_ref, fea2_ref, o_ref):
    m1 = (p1_ref[0] + p1_ref[1]) * (1.0 / _B)
    m2 = (p2_ref[0] + p2_ref[1]) * (1.0 / _B)
    fea_c1 = _M_EMA * m1 + (1.0 - _M_EMA) * fea1_ref[...]
    fea_c2 = _M_EMA * m2 + (1.0 - _M_EMA) * fea2_ref[...]
    z = fea_c1 - jnp.max(fea_c1, axis=1, keepdims=True)
    log_p = z - jnp.log(jnp.sum(jnp.exp(z), axis=1, keepdims=True))
    q = fea_c2 / jnp.sum(fea_c2, axis=1, keepdims=True)
    kl = jnp.where(q > 0, q * (jnp.log(jnp.where(q > 0, q, 1.0)) - log_p), 0.0)
    o_ref[0, 0] = jnp.sum(kl) * (1.0 / _L)


def kernel(f1, f2, pre1, pre2, fea1, fea2):
    f1v = f1.transpose(0, 2, 3, 1).reshape(_B, _H * _W, _C)
    f2v = f2.transpose(0, 2, 3, 1).reshape(_B, _H * _W, _C)

    f_spec = pl.BlockSpec((1, _H * _W, _C), lambda k, i: (k * _BPC + i, 0, 0))
    pre_spec = pl.BlockSpec((1, _L, 2), lambda k, i: (k * _BPC + i, 0, 0))
    out_spec = pl.BlockSpec((1, _L, _C), lambda k, i: (k, 0, 0))

    p1, p2 = pl.pallas_call(
        _pool_kernel,
        grid=(_CORES, _BPC),
        in_specs=[f_spec, f_spec, pre_spec, pre_spec],
        out_specs=[out_spec, out_spec],
        out_shape=[jax.ShapeDtypeStruct((_CORES, _L, _C), jnp.float32)] * 2,
        compiler_params=pltpu.CompilerParams(
            dimension_semantics=("parallel", "arbitrary")),
        name="window_pool",
    )(f1v, f2v, pre1, pre2)

    out = pl.pallas_call(
        _finalize_kernel,
        in_specs=[pl.BlockSpec(memory_space=pltpu.VMEM)] * 4,
        out_specs=pl.BlockSpec(memory_space=pltpu.SMEM),
        out_shape=jax.ShapeDtypeStruct((1, 1), jnp.float32),
        name="ema_kl_finalize",
    )(p1, p2, fea1, fea2)
    return out[0, 0]


# speedup vs baseline: 16.1571x; 1.0303x over previous
import jax
import jax.numpy as jnp
from jax.experimental import pallas as pl
from jax.experimental.pallas import tpu as pltpu

_HALF = 6.0
_MAXC = 63.0
_M_EMA = 0.999
_B, _C, _H, _W, _L = 64, 256, 64, 64, 21
_CORES = 2
_BPC = _B // _CORES


def _pool_one(f_blk, pre_t_ref, onehot):
    x = jnp.dot(pre_t_ref[:, 0, :], onehot,
                preferred_element_type=jnp.float32)
    y = jnp.dot(pre_t_ref[:, 1, :], onehot,
                preferred_element_type=jnp.float32)
    down = jnp.maximum(y - _HALF, 0.0).astype(jnp.int32)
    left = jnp.maximum(x - _HALF, 0.0).astype(jnp.int32)
    upper = jnp.minimum(y + _HALF, _MAXC).astype(jnp.int32)
    right = jnp.minimum(x + _HALF, _MAXC).astype(jnp.int32)
    j = jax.lax.broadcasted_iota(jnp.int32, (_L, _H * _W), 1)
    h = j >> 6
    w = j & (_W - 1)
    mask = ((h >= left) & (h < right) & (w >= down) & (w < upper)
            ).astype(jnp.float32)
    s = ((upper - down + 1) * (right - left + 1)).astype(jnp.float32)
    pooled = jnp.dot(mask, f_blk[0], preferred_element_type=jnp.float32)
    return pooled * (1.0 / s)


def _pool_kernel(f1_ref, f2_ref, pre1_ref, pre2_ref, o1_ref, o2_ref):
    k = pl.program_id(0)
    i = pl.program_id(1)

    @pl.when(i == 0)
    def _():
        o1_ref[...] = jnp.zeros_like(o1_ref)
        o2_ref[...] = jnp.zeros_like(o2_ref)

    b = k * _BPC + i
    bi = jax.lax.broadcasted_iota(jnp.int32, (_B, 1), 0)
    onehot = (bi == b).astype(jnp.float32)
    o1_ref[...] += _pool_one(f1_ref, pre1_ref, onehot)[None]
    o2_ref[...] += _pool_one(f2_ref, pre2_ref, onehot)[None]


def _finalize_kernel(p1_ref, p2_ref, fea1_ref, fea2_ref, o_ref):
    m1 = (p1_ref[0] + p1_ref[1]) * (1.0 / _B)
    m2 = (p2_ref[0] + p2_ref[1]) * (1.0 / _B)
    fea_c1 = _M_EMA * m1 + (1.0 - _M_EMA) * fea1_ref[...]
    fea_c2 = _M_EMA * m2 + (1.0 - _M_EMA) * fea2_ref[...]
    z = fea_c1 - jnp.max(fea_c1, axis=1, keepdims=True)
    log_p = z - jnp.log(jnp.sum(jnp.exp(z), axis=1, keepdims=True))
    q = fea_c2 / jnp.sum(fea_c2, axis=1, keepdims=True)
    kl = jnp.where(q > 0, q * (jnp.log(jnp.where(q > 0, q, 1.0)) - log_p), 0.0)
    o_ref[0, 0] = jnp.sum(kl) * (1.0 / _L)


def kernel(f1, f2, pre1, pre2, fea1, fea2):
    f1v = f1.transpose(0, 2, 3, 1).reshape(_B, _H * _W, _C)
    f2v = f2.transpose(0, 2, 3, 1).reshape(_B, _H * _W, _C)
    pre1t = pre1.transpose(1, 2, 0)
    pre2t = pre2.transpose(1, 2, 0)

    f_spec = pl.BlockSpec((1, _H * _W, _C), lambda k, i: (k * _BPC + i, 0, 0))
    pre_spec = pl.BlockSpec((_L, 2, _B), lambda k, i: (0, 0, 0))
    out_spec = pl.BlockSpec((1, _L, _C), lambda k, i: (k, 0, 0))

    p1, p2 = pl.pallas_call(
        _pool_kernel,
        grid=(_CORES, _BPC),
        in_specs=[f_spec, f_spec, pre_spec, pre_spec],
        out_specs=[out_spec, out_spec],
        out_shape=[jax.ShapeDtypeStruct((_CORES, _L, _C), jnp.float32)] * 2,
        compiler_params=pltpu.CompilerParams(
            dimension_semantics=("parallel", "arbitrary")),
        name="window_pool",
    )(f1v, f2v, pre1t, pre2t)

    out = pl.pallas_call(
        _finalize_kernel,
        in_specs=[pl.BlockSpec(memory_space=pltpu.VMEM)] * 4,
        out_specs=pl.BlockSpec(memory_space=pltpu.SMEM),
        out_shape=jax.ShapeDtypeStruct((1, 1), jnp.float32),
        name="ema_kl_finalize",
    )(p1, p2, fea1, fea2)
    return out[0, 0]


# speedup vs baseline: 16.1628x; 1.0004x over previous
import jax
import jax.numpy as jnp
from jax.experimental import pallas as pl
from jax.experimental.pallas import tpu as pltpu

_HALF = 6.0
_MAXC = 63.0
_M_EMA = 0.999
_B, _C, _H, _W, _L = 64, 256, 64, 64, 21
_CORES = 2
_BPC = _B // _CORES


def _pool_one(f_blk, pre_t_ref, onehot):
    x = jnp.sum(pre_t_ref[:, 0, :] * onehot, axis=1, keepdims=True)
    y = jnp.sum(pre_t_ref[:, 1, :] * onehot, axis=1, keepdims=True)
    down = jnp.maximum(y - _HALF, 0.0).astype(jnp.int32)
    left = jnp.maximum(x - _HALF, 0.0).astype(jnp.int32)
    upper = jnp.minimum(y + _HALF, _MAXC).astype(jnp.int32)
    right = jnp.minimum(x + _HALF, _MAXC).astype(jnp.int32)
    j = jax.lax.broadcasted_iota(jnp.int32, (_L, _H * _W), 1)
    h = j >> 6
    w = j & (_W - 1)
    mask = ((h >= left) & (h < right) & (w >= down) & (w < upper)
            ).astype(jnp.float32)
    s = ((upper - down + 1) * (right - left + 1)).astype(jnp.float32)
    pooled = jnp.dot(mask, f_blk[0], preferred_element_type=jnp.float32)
    return pooled / s


def _pool_kernel(f1_ref, f2_ref, pre1_ref, pre2_ref, o1_ref, o2_ref):
    k = pl.program_id(0)
    i = pl.program_id(1)

    @pl.when(i == 0)
    def _():
        o1_ref[...] = jnp.zeros_like(o1_ref)
        o2_ref[...] = jnp.zeros_like(o2_ref)

    b = k * _BPC + i
    bi = jax.lax.broadcasted_iota(jnp.int32, (1, _B), 1)
    onehot = (bi == b).astype(jnp.float32)
    o1_ref[...] += _pool_one(f1_ref, pre1_ref, onehot)[None]
    o2_ref[...] += _pool_one(f2_ref, pre2_ref, onehot)[None]


def _finalize_kernel(p1_ref, p2_ref, fea1_ref, fea2_ref, o_ref):
    m1 = (p1_ref[0] + p1_ref[1]) * (1.0 / _B)
    m2 = (p2_ref[0] + p2_ref[1]) * (1.0 / _B)
    fea_c1 = _M_EMA * m1 + (1.0 - _M_EMA) * fea1_ref[...]
    fea_c2 = _M_EMA * m2 + (1.0 - _M_EMA) * fea2_ref[...]
    z = fea_c1 - jnp.max(fea_c1, axis=1, keepdims=True)
    log_p = z - jnp.log(jnp.sum(jnp.exp(z), axis=1, keepdims=True))
    q = fea_c2 / jnp.sum(fea_c2, axis=1, keepdims=True)
    kl = jnp.where(q > 0, q * (jnp.log(jnp.where(q > 0, q, 1.0)) - log_p), 0.0)
    o_ref[0, 0] = jnp.sum(kl) * (1.0 / _L)


def kernel(f1, f2, pre1, pre2, fea1, fea2):
    f1v = f1.transpose(0, 2, 3, 1).reshape(_B, _H * _W, _C)
    f2v = f2.transpose(0, 2, 3, 1).reshape(_B, _H * _W, _C)
    pre1t = pre1.transpose(1, 2, 0)
    pre2t = pre2.transpose(1, 2, 0)

    f_spec = pl.BlockSpec((1, _H * _W, _C), lambda k, i: (k * _BPC + i, 0, 0))
    pre_spec = pl.BlockSpec((_L, 2, _B), lambda k, i: (0, 0, 0))
    out_spec = pl.BlockSpec((1, _L, _C), lambda k, i: (k, 0, 0))

    p1, p2 = pl.pallas_call(
        _pool_kernel,
        grid=(_CORES, _BPC),
        in_specs=[f_spec, f_spec, pre_spec, pre_spec],
        out_specs=[out_spec, out_spec],
        out_shape=[jax.ShapeDtypeStruct((_CORES, _L, _C), jnp.float32)] * 2,
        compiler_params=pltpu.CompilerParams(
            dimension_semantics=("parallel", "arbitrary")),
        name="window_pool",
    )(f1v, f2v, pre1t, pre2t)

    out = pl.pallas_call(
        _finalize_kernel,
        in_specs=[pl.BlockSpec(memory_space=pltpu.VMEM)] * 4,
        out_specs=pl.BlockSpec(memory_space=pltpu.SMEM),
        out_shape=jax.ShapeDtypeStruct((1, 1), jnp.float32),
        name="ema_kl_finalize",
    )(p1, p2, fea1, fea2)
    return out[0, 0]
